# Initial kernel scaffold; baseline (speedup 1.0000x reference)
import jax
import jax.numpy as jnp
from jax.experimental import pallas as pl


def kernel(cls_s0, cls_s1, cls_s2, cls_s3, bbox_s0, bbox_s1, bbox_s2, bbox_s3, orig_h, orig_w):
    raise NotImplementedError("write your pallas kernel here")


# speedup vs baseline: 29.3635x; 29.3635x over previous
import jax
import jax.numpy as jnp
from jax import lax
from jax.experimental import pallas as pl
from jax.experimental.pallas import tpu as pltpu

_C = 80
_STRIDES = (8.0, 16.0, 32.0, 64.0)
_HW = (64, 32, 16, 8)
_THR = 0.025
_IOU = 0.6
_K = 1000
_NEG = -1.0e30


def _nms_body(tv0, ti0, bb0, tv1, ti1, bb1, tv2, ti2, bb2, tv3, ti3, bb3,
              out_ref, vr, x1r_, y1r_, x2r_, y2r_, arear, rankr, pr, slotr):
    f32 = jnp.float32
    tvs = (tv0, tv1, tv2, tv3)
    tis = (ti0, ti1, ti2, ti3)
    bbs = (bb0, bb1, bb2, bb3)

    Vl, X1l, Y1l, X2l, Y2l, LBl = [], [], [], [], [], []
    for l in range(4):
        stride = _STRIDES[l]
        H = _HW[l]
        HW = H * H
        tv = tvs[l][:, :]
        ti = tis[l][:, :]
        bb = bbs[l][:, :]

        dists = []
        for k in range(4):
            g = bb[:, 8 * k:8 * k + 8]
            m = jnp.max(g, axis=1, keepdims=True)
            e = jnp.exp(g - m)
            s = jnp.sum(e, axis=1)
            w8 = lax.broadcasted_iota(jnp.int32, (HW, 8), 1).astype(f32)
            num = jnp.sum(e * w8, axis=1)
            dists.append(num / s * stride)

        kq = jnp.floor((ti + 0.5) * (1.0 / _C))
        lb = ti - kq * _C
        rowi = jnp.floor((kq + 0.5) * (1.0 / H))
        coli = kq - rowi * H
        px = (coli + 0.5) * stride
        py = (rowi + 0.5) * stride

        pos_l = (lax.broadcasted_iota(jnp.int32, (8, 128), 0) * 128 +
                 lax.broadcasted_iota(jnp.int32, (8, 128), 1)).astype(f32)
        ispad = pos_l >= float(_K)
        V = jnp.where(ispad, -jnp.inf, tv)

        x1r, y1r, x2r, y2r = [], [], [], []
        for r in range(8):
            kr = kq[r]
            ih = lax.broadcasted_iota(jnp.int32, (HW, 128), 0).astype(f32)
            oh = jnp.where(ih == kr[None, :], 1.0, 0.0)
            d0 = jnp.sum(oh * dists[0][:, None], axis=0)
            d1 = jnp.sum(oh * dists[1][:, None], axis=0)
            d2 = jnp.sum(oh * dists[2][:, None], axis=0)
            d3 = jnp.sum(oh * dists[3][:, None], axis=0)
            x1r.append((px[r] - d0)[None, :])
            y1r.append((py[r] - d1)[None, :])
            x2r.append((px[r] + d2)[None, :])
            y2r.append((py[r] + d3)[None, :])
        X1 = jnp.where(ispad, _NEG, jnp.concatenate(x1r, axis=0))
        Y1 = jnp.where(ispad, _NEG, jnp.concatenate(y1r, axis=0))
        X2 = jnp.where(ispad, _NEG, jnp.concatenate(x2r, axis=0))
        Y2 = jnp.where(ispad, _NEG, jnp.concatenate(y2r, axis=0))
        Vl.append(V); X1l.append(X1); Y1l.append(Y1)
        X2l.append(X2); Y2l.append(Y2); LBl.append(jnp.where(ispad, 0.0, lb))

    V = jnp.concatenate(Vl, axis=0)
    X1 = jnp.concatenate(X1l, axis=0)
    Y1 = jnp.concatenate(Y1l, axis=0)
    X2 = jnp.concatenate(X2l, axis=0)
    Y2 = jnp.concatenate(Y2l, axis=0)
    LB = jnp.concatenate(LBl, axis=0)

    POS = (lax.broadcasted_iota(jnp.int32, (32, 128), 0) * 128 +
           lax.broadcasted_iota(jnp.int32, (32, 128), 1)).astype(f32)
    VALID = jnp.where(V > _THR, 1.0, 0.0)

    gmax = jnp.max(jnp.maximum(jnp.maximum(X1, X2), jnp.maximum(Y1, Y2)))
    off = LB * (gmax + 1.0)
    sx1 = X1 + off
    sy1 = Y1 + off
    sx2 = X2 + off
    sy2 = Y2 + off
    area = jnp.maximum(sx2 - sx1, 0.0) * jnp.maximum(sy2 - sy1, 0.0)

    vr[:, :] = V
    x1r_[:, :] = sx1
    y1r_[:, :] = sy1
    x2r_[:, :] = sx2
    y2r_[:, :] = sy2
    arear[:, :] = area

    lane = lax.broadcasted_iota(jnp.int32, (1, 128), 1).astype(f32)

    def rbody(w, carry):
        vi = vr[pl.ds(w, 1), :][0][:, None, None]
        pi = (jnp.float32(w) * 128.0 + lane)[0][:, None, None]
        gt = (V[None, :, :] > vi)
        eq = (V[None, :, :] == vi) & (POS[None, :, :] < pi)
        cnt = jnp.sum(jnp.sum(jnp.where(gt | eq, 1.0, 0.0), axis=2), axis=1)
        rankr[pl.ds(w, 1), :] = cnt[None, :]
        return carry

    lax.fori_loop(0, 32, rbody, 0)
    RANK = rankr[:, :]

    sh = lax.broadcasted_iota(jnp.int32, (1, 32, 1), 1)

    def pbody(w, carry):
        xi1 = x1r_[pl.ds(w, 1), :][0][:, None, None]
        yi1 = y1r_[pl.ds(w, 1), :][0][:, None, None]
        xi2 = x2r_[pl.ds(w, 1), :][0][:, None, None]
        yi2 = y2r_[pl.ds(w, 1), :][0][:, None, None]
        ai = arear[pl.ds(w, 1), :][0][:, None, None]
        ri = rankr[pl.ds(w, 1), :][0][:, None, None]
        ix1 = jnp.maximum(xi1, sx1[None, :, :])
        iy1 = jnp.maximum(yi1, sy1[None, :, :])
        ix2 = jnp.minimum(xi2, sx2[None, :, :])
        iy2 = jnp.minimum(yi2, sy2[None, :, :])
        inter = jnp.maximum(ix2 - ix1, 0.0) * jnp.maximum(iy2 - iy1, 0.0)
        union = ai + area[None, :, :] - inter
        iou = inter / jnp.maximum(union, 1e-9)
        cond = (iou > _IOU) & (RANK[None, :, :] < ri)
        pblk = jnp.sum(lax.shift_left(cond.astype(jnp.int32), sh), axis=1)
        pr[pl.ds(w, 1), :, :] = pblk[None]
        return carry

    lax.fori_loop(0, 32, pbody, 0)
    P = pr[:, :, :]

    shw = lax.broadcasted_iota(jnp.int32, (32, 128), 0)

    def wcond(st):
        return st[1]

    def wbody(st):
        keep, _ = st
        kw = jnp.sum(lax.shift_left(keep.astype(jnp.int32), shw), axis=0)
        a = P & kw[None, None, :]
        ne = jnp.where(a != 0, 1.0, 0.0)
        sup = jnp.max(ne, axis=2)
        newk = VALID * (1.0 - sup)
        ch = jnp.max(jnp.abs(newk - keep)) > 0.0
        return (newk, ch)

    keep, _ = lax.while_loop(wcond, wbody, (VALID, jnp.bool_(True)))

    def sbody(w, carry):
        ri = rankr[pl.ds(w, 1), :][0][:, None, None]
        lt = jnp.where(RANK[None, :, :] < ri, 1.0, 0.0)
        cnt = jnp.sum(jnp.sum(keep[None, :, :] * lt, axis=2), axis=1)
        slotr[pl.ds(w, 1), :] = cnt[None, :]
        return carry

    lax.fori_loop(0, 32, sbody, 0)
    SLOT = slotr[:, :]
    ktot = jnp.sum(keep)

    sB = lax.broadcasted_iota(jnp.int32, (128, 32, 128), 0).astype(f32)
    hit = keep[None, :, :] * jnp.where(SLOT[None, :, :] == sB, 1.0, 0.0)
    fill = jnp.where((sB >= ktot) & (RANK[None, :, :] == 0.0), 1.0, 0.0)
    O = hit + fill

    Vout = jnp.maximum(V, _NEG)
    rows = []
    for ch in (LB, X1, Y1, X2, Y2, Vout):
        rows.append(jnp.sum(jnp.sum(O * ch[None, :, :], axis=2), axis=1)[None, :])
    rows.append(jnp.zeros((2, 128), f32))
    out_ref[:, :] = jnp.concatenate(rows, axis=0)


def _prep_level(cls, l):
    n = _HW[l] * _HW[l] * _C
    c = cls[0].transpose(1, 2, 0).reshape(n)
    sc = jax.nn.sigmoid(c)
    masked = jnp.where(sc > _THR, sc, -1.0)
    tv, ti = lax.top_k(masked, _K)
    tv = jnp.pad(tv, (0, 1024 - _K)).reshape(8, 128)
    ti = jnp.pad(ti, (0, 1024 - _K)).astype(jnp.float32).reshape(8, 128)
    return tv, ti


def kernel(cls_s0, cls_s1, cls_s2, cls_s3, bbox_s0, bbox_s1, bbox_s2, bbox_s3, orig_h, orig_w):
    clss = (cls_s0, cls_s1, cls_s2, cls_s3)
    bbs = (bbox_s0, bbox_s1, bbox_s2, bbox_s3)
    args = []
    for l in range(4):
        tv, ti = _prep_level(clss[l], l)
        hw = _HW[l] * _HW[l]
        bbf = bbs[l][0].transpose(1, 2, 0).reshape(hw, 32)
        args += [tv, ti, bbf]

    out = pl.pallas_call(
        _nms_body,
        out_shape=jax.ShapeDtypeStruct((8, 128), jnp.float32),
        scratch_shapes=[
            pltpu.VMEM((32, 128), jnp.float32),
            pltpu.VMEM((32, 128), jnp.float32),
            pltpu.VMEM((32, 128), jnp.float32),
            pltpu.VMEM((32, 128), jnp.float32),
            pltpu.VMEM((32, 128), jnp.float32),
            pltpu.VMEM((32, 128), jnp.float32),
            pltpu.VMEM((32, 128), jnp.float32),
            pltpu.VMEM((32, 128, 128), jnp.int32),
            pltpu.VMEM((32, 128), jnp.float32),
        ],
    )(*args)

    labels = out[0, :100].astype(jnp.int32)
    in_w = float(cls_s0.shape[-1]) * 8.0
    in_h = float(cls_s0.shape[-2]) * 8.0
    scale = jnp.stack([orig_w / in_w, orig_h / in_h,
                       orig_w / in_w, orig_h / in_h]).astype(jnp.float32)
    boxes = jnp.transpose(out[1:5, :100]) * scale[None, :]
    scores = out[5, :100]
    return (labels, boxes, scores)
